# baseline (speedup 1.0000x reference)
import functools

import jax
import jax.numpy as jnp
from jax.experimental import pallas as pl
from jax.experimental.pallas import tpu as pltpu


def _enhancer_kernel(w_ref, b_ref, x_ref, o_ref, *, T, BB):
    L = BB * T
    t = jax.lax.broadcasted_iota(jnp.int32, (1, L), 1)
    tm = jax.lax.rem(t, T)
    m_first = tm == 0
    m_last = tm == T - 1
    zero = jnp.bfloat16(0)

    xs = jnp.concatenate(
        [x_ref[i].astype(jnp.bfloat16) for i in range(BB)], axis=1)
    x_prev = jnp.where(m_first, zero, pltpu.roll(xs, shift=1, axis=1))
    x_next = jnp.where(m_last, zero, pltpu.roll(xs, shift=L - 1, axis=1))

    stack = jnp.concatenate([x_prev, xs, x_next], axis=0)
    y = jnp.dot(w_ref[...], stack, preferred_element_type=jnp.float32)

    bias = b_ref[...]
    for i in range(BB):
        o_ref[i] = y[:, i * T:(i + 1) * T] + bias


def kernel(mel_spec, w_taps, bias_col):
    B, M, T = mel_spec.shape
    BB = 8
    w_folded = w_taps.at[1].add(jnp.float32(0.7) * jnp.eye(M, dtype=w_taps.dtype))
    w_cat = jnp.concatenate(
        [w_folded[0], w_folded[1], w_folded[2]], axis=1).astype(jnp.bfloat16)

    return pl.pallas_call(
        functools.partial(_enhancer_kernel, T=T, BB=BB),
        out_shape=jax.ShapeDtypeStruct((B, M, T), mel_spec.dtype),
        grid=(B // BB,),
        in_specs=[
            pl.BlockSpec((M, 3 * M), lambda b: (0, 0)),
            pl.BlockSpec((M, 1), lambda b: (0, 0)),
            pl.BlockSpec((BB, M, T), lambda b: (b, 0, 0)),
        ],
        out_specs=pl.BlockSpec((BB, M, T), lambda b: (b, 0, 0)),
        compiler_params=pltpu.CompilerParams(
            dimension_semantics=("parallel",),
            vmem_limit_bytes=64 << 20,
        ),
    )(w_cat, bias_col, mel_spec)


# speedup vs baseline: 1.0176x; 1.0176x over previous
import functools

import jax
import jax.numpy as jnp
from jax.experimental import pallas as pl
from jax.experimental.pallas import tpu as pltpu


def _enhancer_kernel(w_ref, b_ref, x_ref, o_ref, *, T, BB):
    L = BB * T
    t = jax.lax.broadcasted_iota(jnp.int32, (1, L), 1)
    tm = jax.lax.rem(t, T)
    m_first = tm == 0
    m_last = tm == T - 1
    zero = jnp.bfloat16(0)

    xs = jnp.concatenate(
        [x_ref[i].astype(jnp.bfloat16) for i in range(BB)], axis=1)
    x_prev = jnp.where(m_first, zero, pltpu.roll(xs, shift=1, axis=1))
    x_next = jnp.where(m_last, zero, pltpu.roll(xs, shift=L - 1, axis=1))

    y = jnp.dot(w_ref[0], x_prev, preferred_element_type=jnp.float32)
    y = y + jnp.dot(w_ref[1], xs, preferred_element_type=jnp.float32)
    y = y + jnp.dot(w_ref[2], x_next, preferred_element_type=jnp.float32)

    bias = b_ref[...]
    for i in range(BB):
        o_ref[i] = y[:, i * T:(i + 1) * T] + bias


def kernel(mel_spec, w_taps, bias_col):
    B, M, T = mel_spec.shape
    BB = 8
    w_folded = w_taps.at[1].add(jnp.float32(0.7) * jnp.eye(M, dtype=w_taps.dtype))
    w_bf16 = w_folded.astype(jnp.bfloat16)

    return pl.pallas_call(
        functools.partial(_enhancer_kernel, T=T, BB=BB),
        out_shape=jax.ShapeDtypeStruct((B, M, T), mel_spec.dtype),
        grid=(B // BB,),
        in_specs=[
            pl.BlockSpec((3, M, M), lambda b: (0, 0, 0)),
            pl.BlockSpec((M, 1), lambda b: (0, 0)),
            pl.BlockSpec((BB, M, T), lambda b: (b, 0, 0)),
        ],
        out_specs=pl.BlockSpec((BB, M, T), lambda b: (b, 0, 0)),
        compiler_params=pltpu.CompilerParams(
            dimension_semantics=("parallel",),
            vmem_limit_bytes=64 << 20,
        ),
    )(w_bf16, bias_col, mel_spec)
